# baseline (speedup 1.0000x reference)
import jax
import jax.numpy as jnp
from jax import lax
from jax.experimental import pallas as pl
from jax.experimental.pallas import tpu as pltpu
from jax.experimental.pallas import tpu_sc as plsc

B = 16384
K = 32
L = 4
NU = 1000000
NC, NS, LANES = 2, 16, 16
NW = NC * NS
BPW = B // NW
NG = BPW // LANES


def _sc_body(user_hbm, codes_hbm, uf_hbm, ub_hbm,
             e0_hbm, e1_hbm, e2_hbm, e3_hbm, bias_hbm, params_hbm,
             out_hbm,
             idx_v, codes_v, uf_v, ub_v, rows_v, bias_v, params_v, out_v, sem):
    wid = lax.axis_index("s") * NC + lax.axis_index("c")
    base = wid * BPW

    pltpu.sync_copy(user_hbm.at[pl.ds(base, BPW)], idx_v)
    pltpu.sync_copy(codes_hbm.at[wid], codes_v)
    pltpu.sync_copy(bias_hbm, bias_v)
    pltpu.sync_copy(params_hbm, params_v)

    cps = [pltpu.async_copy(ub_hbm.at[idx_v], ub_v, sem)]
    for k in range(K):
        cps.append(pltpu.async_copy(
            uf_hbm.at[pl.ds(k * NU, NU)].at[idx_v], uf_v.at[k], sem))
    for l, e_hbm in enumerate((e0_hbm, e1_hbm, e2_hbm, e3_hbm)):
        cps.append(pltpu.async_copy(e_hbm.at[codes_v.at[l]], rows_v.at[l], sem))

    lane = lax.iota(jnp.int32, LANES)
    lv = params_v[0, :]
    muv = params_v[1, :]
    valid = lane < L
    def _dyn_gather(a, idx):
        dnums = lax.GatherDimensionNumbers(
            offset_dims=(), collapsed_slice_dims=(0,), start_index_map=(0,))
        return lax.gather(a, idx[:, None], dnums, (1,),
                          mode=lax.GatherScatterMode.PROMISE_IN_BOUNDS)

    def _splat_reduce(a, comb):
        for sh in (8, 4, 2, 1):
            a = comb(a, _dyn_gather(a, (lane + sh) & (LANES - 1)))
        return a

    m = _splat_reduce(jnp.where(valid, lv, 0.0), jnp.maximum)
    e = jnp.where(valid, jnp.exp(lv - m), 0.0)
    w = e / _splat_reduce(e, jnp.add)
    wl = [_dyn_gather(w, jnp.full((LANES,), l, jnp.int32)) for l in range(L)]

    for cp in cps:
        cp.wait()

    def g_body(g, _):
        row = g * LANES + lane
        acc = muv + plsc.load_gather(ub_v, [row])
        for l in range(L):
            cl = codes_v[l, pl.ds(g * LANES, LANES)]
            lvl = jnp.full((LANES,), l, jnp.int32)
            acc = acc + wl[l] * plsc.load_gather(bias_v, [lvl, cl])

        def k_body(k, a):
            col = jnp.zeros((LANES,), jnp.int32) + k
            u = uf_v[k, pl.ds(g * LANES, LANES)]
            vsum = wl[0] * plsc.load_gather(
                rows_v, [jnp.full((LANES,), 0, jnp.int32), row, col])
            for l in range(1, L):
                lvl = jnp.full((LANES,), l, jnp.int32)
                vsum = vsum + wl[l] * plsc.load_gather(rows_v, [lvl, row, col])
            return a + u * vsum

        acc = lax.fori_loop(0, K, k_body, acc)
        out_v[pl.ds(g * LANES, LANES)] = acc
        return 0

    lax.fori_loop(0, NG, g_body, 0)
    pltpu.sync_copy(out_v, out_hbm.at[pl.ds(base, BPW)])


def kernel(user, sem_codes, user_f, user_b,
           sem_emb_0, sem_emb_1, sem_emb_2, sem_emb_3,
           sem_bias_0, sem_bias_1, sem_bias_2, sem_bias_3,
           level_logits, mu):
    embs = (sem_emb_0, sem_emb_1, sem_emb_2, sem_emb_3)
    biases = (sem_bias_0, sem_bias_1, sem_bias_2, sem_bias_3)

    cols = [jnp.clip(sem_codes[:, l], 0, embs[l].shape[0] - 1)
            for l in range(L)]
    codes_w = jnp.stack(cols).reshape(L, NW, BPW).transpose(1, 0, 2)

    bias_stack = jnp.stack([b[:, 0] for b in biases])
    params = jnp.stack([
        jnp.concatenate([level_logits,
                         jnp.zeros((LANES - L,), jnp.float32)]),
        jnp.broadcast_to(mu, (LANES,)).astype(jnp.float32),
    ])

    run = pl.kernel(
        _sc_body,
        out_type=jax.ShapeDtypeStruct((B,), jnp.float32),
        mesh=plsc.VectorSubcoreMesh(core_axis_name="c", subcore_axis_name="s",
                                    num_cores=NC, num_subcores=NS),
        compiler_params=pltpu.CompilerParams(needs_layout_passes=False,
                                             use_tc_tiling_on_sc=False),
        scratch_types=[
            pltpu.VMEM((BPW,), jnp.int32),
            pltpu.VMEM((L, BPW), jnp.int32),
            pltpu.VMEM((K, BPW), jnp.float32),
            pltpu.VMEM((BPW,), jnp.float32),
            pltpu.VMEM((L, BPW, K), jnp.float32),
            pltpu.VMEM((L, 1024), jnp.float32),
            pltpu.VMEM((2, LANES), jnp.float32),
            pltpu.VMEM((BPW,), jnp.float32),
            pltpu.SemaphoreType.DMA,
        ],
    )
    uf_cm = user_f.T.reshape(K * NU)
    return run(user, codes_w, uf_cm, user_b[:, 0],
               sem_emb_0, sem_emb_1, sem_emb_2, sem_emb_3,
               bias_stack, params)


# speedup vs baseline: 4.7176x; 4.7176x over previous
import jax
import jax.numpy as jnp
from jax import lax
from jax.experimental import pallas as pl
from jax.experimental.pallas import tpu as pltpu
from jax.experimental.pallas import tpu_sc as plsc

B = 16384
K = 32
KP = 128
L = 4
NU = 1000000
NC, NS, LANES = 2, 16, 16
NW = NC * NS
BPW = B // NW
HALF = BPW // 2
NGH = HALF // LANES


def _sc_body(user_hbm, codes_hbm, ufp_hbm, ub_hbm,
             e0_hbm, e1_hbm, e2_hbm, e3_hbm, bias_hbm, params_hbm,
             out_hbm,
             idx_v, codes_v, ufp_v, ub_v, rows_v, bias_v, params_v, out_v,
             sem):
    wid = lax.axis_index("s") * NC + lax.axis_index("c")
    base = wid * BPW

    pltpu.sync_copy(user_hbm.at[pl.ds(base, BPW)], idx_v)
    pltpu.sync_copy(codes_hbm.at[wid], codes_v)
    pltpu.sync_copy(bias_hbm, bias_v)
    pltpu.sync_copy(params_hbm, params_v)

    cps = [pltpu.async_copy(ub_hbm.at[idx_v], ub_v, sem)]
    for l, e_hbm in enumerate((e0_hbm, e1_hbm, e2_hbm, e3_hbm)):
        cps.append(pltpu.async_copy(e_hbm.at[codes_v.at[l]], rows_v.at[l], sem))
    cp_uf = pltpu.async_copy(ufp_hbm.at[codes_v.at[L, pl.ds(0, HALF)]],
                             ufp_v, sem)

    lane = lax.iota(jnp.int32, LANES)
    lv = params_v[0, :]
    muv = params_v[1, :]
    valid = lane < L

    def _dyn_gather(a, idx):
        dnums = lax.GatherDimensionNumbers(
            offset_dims=(), collapsed_slice_dims=(0,), start_index_map=(0,))
        return lax.gather(a, idx[:, None], dnums, (1,),
                          mode=lax.GatherScatterMode.PROMISE_IN_BOUNDS)

    def _splat_reduce(a, comb):
        for sh in (8, 4, 2, 1):
            a = comb(a, _dyn_gather(a, (lane + sh) & (LANES - 1)))
        return a

    m = _splat_reduce(jnp.where(valid, lv, 0.0), jnp.maximum)
    e = jnp.where(valid, jnp.exp(lv - m), 0.0)
    w = e / _splat_reduce(e, jnp.add)
    wl = [_dyn_gather(w, jnp.full((LANES,), l, jnp.int32)) for l in range(L)]

    for cp in cps:
        cp.wait()

    def _half_body(h):
        def g_body(g, _):
            grow = h * HALF + g * LANES + lane
            lrow = g * LANES + lane
            acc = muv + plsc.load_gather(ub_v, [grow])
            for l in range(L):
                cl = codes_v[l, pl.ds(h * HALF + g * LANES, LANES)]
                lvl = jnp.full((LANES,), l, jnp.int32)
                acc = acc + wl[l] * plsc.load_gather(bias_v, [lvl, cl])

            ucol = codes_v[L + 1, pl.ds(h * HALF + g * LANES, LANES)]

            def k_body(k, a):
                col = jnp.zeros((LANES,), jnp.int32) + k
                u = plsc.load_gather(ufp_v, [lrow, ucol + col])
                vsum = wl[0] * plsc.load_gather(
                    rows_v, [jnp.full((LANES,), 0, jnp.int32), grow, col])
                for l in range(1, L):
                    lvl = jnp.full((LANES,), l, jnp.int32)
                    vsum = vsum + wl[l] * plsc.load_gather(
                        rows_v, [lvl, grow, col])
                return a + u * vsum

            acc = lax.fori_loop(0, K, k_body, acc)
            out_v[pl.ds(h * HALF + g * LANES, LANES)] = acc
            return 0
        return g_body

    cp_uf.wait()
    lax.fori_loop(0, NGH, _half_body(0), 0)
    cp_uf2 = pltpu.async_copy(ufp_hbm.at[codes_v.at[L, pl.ds(HALF, HALF)]],
                              ufp_v, sem)
    cp_uf2.wait()
    lax.fori_loop(0, NGH, _half_body(1), 0)
    pltpu.sync_copy(out_v, out_hbm.at[pl.ds(base, BPW)])


def kernel(user, sem_codes, user_f, user_b,
           sem_emb_0, sem_emb_1, sem_emb_2, sem_emb_3,
           sem_bias_0, sem_bias_1, sem_bias_2, sem_bias_3,
           level_logits, mu):
    embs = (sem_emb_0, sem_emb_1, sem_emb_2, sem_emb_3)
    biases = (sem_bias_0, sem_bias_1, sem_bias_2, sem_bias_3)

    cols = [jnp.clip(sem_codes[:, l], 0, embs[l].shape[0] - 1)
            for l in range(L)]
    cols.append(user // 4)
    cols.append((user % 4) * K)
    codes_w = jnp.stack(cols).reshape(L + 2, NW, BPW).transpose(1, 0, 2)

    bias_stack = jnp.stack([b[:, 0] for b in biases])
    params = jnp.stack([
        jnp.concatenate([level_logits,
                         jnp.zeros((LANES - L,), jnp.float32)]),
        jnp.broadcast_to(mu, (LANES,)).astype(jnp.float32),
    ])

    uf_pack = user_f.reshape(NU // 4, KP)

    run = pl.kernel(
        _sc_body,
        out_type=jax.ShapeDtypeStruct((B,), jnp.float32),
        mesh=plsc.VectorSubcoreMesh(core_axis_name="c", subcore_axis_name="s",
                                    num_cores=NC, num_subcores=NS),
        compiler_params=pltpu.CompilerParams(needs_layout_passes=False,
                                             use_tc_tiling_on_sc=False),
        scratch_types=[
            pltpu.VMEM((BPW,), jnp.int32),
            pltpu.VMEM((L + 2, BPW), jnp.int32),
            pltpu.VMEM((HALF, KP), jnp.float32),
            pltpu.VMEM((BPW,), jnp.float32),
            pltpu.VMEM((L, BPW, K), jnp.float32),
            pltpu.VMEM((L, 1024), jnp.float32),
            pltpu.VMEM((2, LANES), jnp.float32),
            pltpu.VMEM((BPW,), jnp.float32),
            pltpu.SemaphoreType.DMA,
        ],
    )
    return run(user, codes_w, uf_pack, user_b[:, 0],
               sem_emb_0, sem_emb_1, sem_emb_2, sem_emb_3,
               bias_stack, params)


# speedup vs baseline: 4.7952x; 1.0164x over previous
import jax
import jax.numpy as jnp
from jax import lax
from jax.experimental import pallas as pl
from jax.experimental.pallas import tpu as pltpu
from jax.experimental.pallas import tpu_sc as plsc

B = 16384
K = 32
L = 4
NC, NS, LANES = 2, 16, 16
NW = NC * NS
BPW = B // NW
NG = BPW // LANES


def _sc_body(user_hbm, codes_hbm, uf_hbm, ub_hbm,
             e0_hbm, e1_hbm, e2_hbm, e3_hbm, bias_hbm, params_hbm,
             out_hbm,
             idx_v, codes_v, uf_v, ub_v, rows_v, bias_v, params_v, out_v, sem):
    wid = lax.axis_index("s") * NC + lax.axis_index("c")
    base = wid * BPW

    pltpu.sync_copy(user_hbm.at[pl.ds(base, BPW)], idx_v)
    pltpu.sync_copy(codes_hbm.at[wid], codes_v)
    pltpu.sync_copy(bias_hbm, bias_v)
    pltpu.sync_copy(params_hbm, params_v)

    cps = [
        pltpu.async_copy(uf_hbm.at[idx_v], uf_v, sem),
        pltpu.async_copy(ub_hbm.at[idx_v], ub_v, sem),
    ]
    for l, e_hbm in enumerate((e0_hbm, e1_hbm, e2_hbm, e3_hbm)):
        cps.append(pltpu.async_copy(e_hbm.at[codes_v.at[l]], rows_v.at[l], sem))

    lane = lax.iota(jnp.int32, LANES)
    lv = params_v[0, :]
    muv = params_v[1, :]
    valid = lane < L

    def _dyn_gather(a, idx):
        dnums = lax.GatherDimensionNumbers(
            offset_dims=(), collapsed_slice_dims=(0,), start_index_map=(0,))
        return lax.gather(a, idx[:, None], dnums, (1,),
                          mode=lax.GatherScatterMode.PROMISE_IN_BOUNDS)

    def _splat_reduce(a, comb):
        for sh in (8, 4, 2, 1):
            a = comb(a, _dyn_gather(a, (lane + sh) & (LANES - 1)))
        return a

    m = _splat_reduce(jnp.where(valid, lv, 0.0), jnp.maximum)
    e = jnp.where(valid, jnp.exp(lv - m), 0.0)
    w = e / _splat_reduce(e, jnp.add)
    wl = [_dyn_gather(w, jnp.full((LANES,), l, jnp.int32)) for l in range(L)]

    for cp in cps:
        cp.wait()

    lvls = [jnp.full((LANES,), l, jnp.int32) for l in range(L)]

    def g_body(g, _):
        row = g * LANES + lane
        acc = muv + plsc.load_gather(ub_v, [row])
        for l in range(L):
            cl = codes_v[l, pl.ds(g * LANES, LANES)]
            acc = acc + wl[l] * plsc.load_gather(bias_v, [lvls[l], cl])

        for k in range(K):
            col = jnp.full((LANES,), k, jnp.int32)
            u = plsc.load_gather(uf_v, [row, col])
            vsum = wl[0] * plsc.load_gather(rows_v, [lvls[0], row, col])
            for l in range(1, L):
                vsum = vsum + wl[l] * plsc.load_gather(rows_v,
                                                       [lvls[l], row, col])
            acc = acc + u * vsum

        out_v[pl.ds(g * LANES, LANES)] = acc
        return 0

    lax.fori_loop(0, NG, g_body, 0)
    pltpu.sync_copy(out_v, out_hbm.at[pl.ds(base, BPW)])


def kernel(user, sem_codes, user_f, user_b,
           sem_emb_0, sem_emb_1, sem_emb_2, sem_emb_3,
           sem_bias_0, sem_bias_1, sem_bias_2, sem_bias_3,
           level_logits, mu):
    embs = (sem_emb_0, sem_emb_1, sem_emb_2, sem_emb_3)
    biases = (sem_bias_0, sem_bias_1, sem_bias_2, sem_bias_3)

    cols = [jnp.clip(sem_codes[:, l], 0, embs[l].shape[0] - 1)
            for l in range(L)]
    codes_w = jnp.stack(cols).reshape(L, NW, BPW).transpose(1, 0, 2)

    bias_stack = jnp.stack([b[:, 0] for b in biases])
    params = jnp.stack([
        jnp.concatenate([level_logits,
                         jnp.zeros((LANES - L,), jnp.float32)]),
        jnp.broadcast_to(mu, (LANES,)).astype(jnp.float32),
    ])

    run = pl.kernel(
        _sc_body,
        out_type=jax.ShapeDtypeStruct((B,), jnp.float32),
        mesh=plsc.VectorSubcoreMesh(core_axis_name="c", subcore_axis_name="s",
                                    num_cores=NC, num_subcores=NS),
        compiler_params=pltpu.CompilerParams(needs_layout_passes=False,
                                             use_tc_tiling_on_sc=False),
        scratch_types=[
            pltpu.VMEM((BPW,), jnp.int32),
            pltpu.VMEM((L, BPW), jnp.int32),
            pltpu.VMEM((BPW, K), jnp.float32),
            pltpu.VMEM((BPW,), jnp.float32),
            pltpu.VMEM((L, BPW, K), jnp.float32),
            pltpu.VMEM((L, 1024), jnp.float32),
            pltpu.VMEM((2, LANES), jnp.float32),
            pltpu.VMEM((BPW,), jnp.float32),
            pltpu.SemaphoreType.DMA,
        ],
    )
    return run(user, codes_w, user_f, user_b[:, 0],
               sem_emb_0, sem_emb_1, sem_emb_2, sem_emb_3,
               bias_stack, params)
